# Initial kernel scaffold; baseline (speedup 1.0000x reference)
import jax
import jax.numpy as jnp
from jax import lax
from jax.experimental import pallas as pl
from jax.experimental.pallas import tpu as pltpu
from jax.experimental.pallas import tpu_sc as plsc

NC = 2
NS = 16
NW = NC * NS
L = 16
C = 128


def _tc_transform(x, Wl, Wr, block_rows):
    n, d = x.shape
    h = Wl.shape[1]
    grid = n // block_rows

    def body(x_ref, wl_ref, wr_ref, xl_ref, xr_ref):
        xv = x_ref[...]
        xl_ref[...] = jnp.dot(xv, wl_ref[...], preferred_element_type=jnp.float32)
        xr_ref[...] = jnp.dot(xv, wr_ref[...], preferred_element_type=jnp.float32)

    return pl.pallas_call(
        body,
        grid=(grid,),
        in_specs=[
            pl.BlockSpec((block_rows, d), lambda i: (i, 0)),
            pl.BlockSpec((d, h), lambda i: (0, 0)),
            pl.BlockSpec((d, h), lambda i: (0, 0)),
        ],
        out_specs=[
            pl.BlockSpec((block_rows, h), lambda i: (i, 0)),
            pl.BlockSpec((block_rows, h), lambda i: (i, 0)),
        ],
        out_shape=[
            jax.ShapeDtypeStruct((n, h), jnp.float32),
            jax.ShapeDtypeStruct((n, h), jnp.float32),
        ],
    )(x, Wl, Wr)


def _make_sc_layer1(nseg, chunks, etot, h):
    rows_per_tile = nseg // NS
    nvr = h // L

    def body(xl_hbm, xr_hbm, src_hbm, dst_hbm, att_hbm,
             num_out, den_out,
             src_v, dst_v, xl_v, xr_v, num_v, den_v, att_v,
             dacc_v, dtmp_v, num_sh, den_sh, sem1, sem2):
        cid = lax.axis_index("c")
        sid = lax.axis_index("s")
        wid = cid * NS + sid
        zero16 = jnp.zeros((L,), jnp.float32)
        lane = lax.iota(jnp.int32, L)
        mask0 = lane == 0

        def _zrow(i, _):
            for r in range(nvr):
                num_v[i, pl.ds(r * L, L)] = zero16
            return 0
        lax.fori_loop(0, C, _zrow, 0)
        for t in range(rows_per_tile // C):
            pltpu.sync_copy(num_v, num_sh.at[pl.ds(sid * rows_per_tile + t * C, C)])

        def _zden(i, _):
            den_v[pl.ds(i * L, L)] = zero16
            return 0
        lax.fori_loop(0, nseg // L, _zden, 0)

        pltpu.sync_copy(att_hbm, att_v)
        plsc.subcore_barrier()

        attv = [att_v[pl.ds(r * L, L)] for r in range(nvr)]

        def chunk_body(k, _):
            base = wid * (chunks * C) + k * C
            pltpu.sync_copy(src_hbm.at[pl.ds(base, C)], src_v)
            pltpu.sync_copy(dst_hbm.at[pl.ds(base, C)], dst_v)
            d1 = pltpu.async_copy(xl_hbm.at[src_v], xl_v, sem1)
            d2 = pltpu.async_copy(xr_hbm.at[dst_v], xr_v, sem2)
            d1.wait()
            d2.wait()

            def edge_body(j, _):
                a = [xl_v[j, pl.ds(r * L, L)] for r in range(nvr)]
                acc = zero16
                for r in range(nvr):
                    v = a[r] + xr_v[j, pl.ds(r * L, L)]
                    lr = jnp.maximum(v, 0.2 * v)
                    acc = acc + lr * attv[r]
                e = jnp.sum(acc)
                valid = jnp.where(base + j < etot, 1.0, 0.0)
                p = jnp.exp(jnp.full((L,), e, jnp.float32)) * valid
                for r in range(nvr):
                    num_v[j, pl.ds(r * L, L)] = p * a[r]
                dj = dst_v[j]
                plsc.addupdate_scatter(
                    den_v, [jnp.full((L,), dj, jnp.int32)], p, mask=mask0)
                return 0
            lax.fori_loop(0, C, edge_body, 0)
            pltpu.sync_copy(num_v, num_sh.at[dst_v], add=True)
            return 0
        lax.fori_loop(0, chunks, chunk_body, 0)

        plsc.subcore_barrier()
        pltpu.sync_copy(den_v, den_sh.at[sid])
        plsc.subcore_barrier()
        col = sid * rows_per_tile
        pltpu.sync_copy(den_sh.at[0, pl.ds(col, rows_per_tile)], dacc_v)
        for t in range(1, NS):
            pltpu.sync_copy(den_sh.at[t, pl.ds(col, rows_per_tile)], dtmp_v)

            def _acc(i, _):
                dacc_v[pl.ds(i * L, L)] = (
                    dacc_v[pl.ds(i * L, L)] + dtmp_v[pl.ds(i * L, L)])
                return 0
            lax.fori_loop(0, rows_per_tile // L, _acc, 0)
        pltpu.sync_copy(dacc_v, den_out.at[cid, pl.ds(col, rows_per_tile)])
        pltpu.sync_copy(num_sh.at[pl.ds(col, rows_per_tile)],
                        num_out.at[cid, pl.ds(col, rows_per_tile)])

    mesh = plsc.VectorSubcoreMesh(
        core_axis_name="c", subcore_axis_name="s", num_cores=NC, num_subcores=NS)
    return pl.kernel(
        body,
        out_type=[
            jax.ShapeDtypeStruct((NC, nseg, h), jnp.float32),
            jax.ShapeDtypeStruct((NC, nseg), jnp.float32),
        ],
        mesh=mesh,
        scratch_types=[
            pltpu.VMEM((C,), jnp.int32),
            pltpu.VMEM((C,), jnp.int32),
            pltpu.VMEM((C, h), jnp.float32),
            pltpu.VMEM((C, h), jnp.float32),
            pltpu.VMEM((C, h), jnp.float32),
            pltpu.VMEM((nseg,), jnp.float32),
            pltpu.VMEM((h,), jnp.float32),
            pltpu.VMEM((rows_per_tile,), jnp.float32),
            pltpu.VMEM((rows_per_tile,), jnp.float32),
            pltpu.VMEM_SHARED((nseg, h), jnp.float32),
            pltpu.VMEM_SHARED((NS, nseg), jnp.float32),
            pltpu.SemaphoreType.DMA,
            pltpu.SemaphoreType.DMA,
        ],
    )


def _make_sc_layer2(nseg, chunks, etot):
    rows_per_tile = nseg // NS

    def body(xl_hbm, xr_hbm, src_hbm, dst_hbm, att_hbm,
             num_out, den_out,
             src_v, dst_v, xl_v, xr_v, num_v, den_v, att_v,
             acc_v, tmp_v, num_sh, den_sh):
        cid = lax.axis_index("c")
        sid = lax.axis_index("s")
        wid = cid * NS + sid
        zero16 = jnp.zeros((L,), jnp.float32)
        lane = lax.iota(jnp.int32, L)

        pltpu.sync_copy(xl_hbm, xl_v)
        pltpu.sync_copy(xr_hbm, xr_v)
        pltpu.sync_copy(att_hbm, att_v)

        def _z(i, _):
            num_v[pl.ds(i * L, L)] = zero16
            den_v[pl.ds(i * L, L)] = zero16
            return 0
        lax.fori_loop(0, nseg // L, _z, 0)
        att2 = att_v[...]

        def chunk_body(k, _):
            base = wid * (chunks * C) + k * C
            pltpu.sync_copy(src_hbm.at[pl.ds(base, C)], src_v)
            pltpu.sync_copy(dst_hbm.at[pl.ds(base, C)], dst_v)
            for g in range(C // L):
                s16 = src_v[pl.ds(g * L, L)]
                d16 = dst_v[pl.ds(g * L, L)]
                a = plsc.load_gather(xl_v, [s16])
                b = plsc.load_gather(xr_v, [d16])
                v = a + b
                lr = jnp.maximum(v, 0.2 * v)
                p = jnp.exp(lr * att2)
                validv = jnp.where(base + g * L + lane < etot, 1.0, 0.0)
                p = p * validv
                pa = p * a
                for j in range(L):
                    dj = dst_v[g * L + j]
                    idxs = jnp.full((L,), dj, jnp.int32)
                    mj = lane == j
                    plsc.addupdate_scatter(den_v, [idxs], p, mask=mj)
                    plsc.addupdate_scatter(num_v, [idxs], pa, mask=mj)
            return 0
        lax.fori_loop(0, chunks, chunk_body, 0)

        pltpu.sync_copy(num_v, num_sh.at[sid])
        pltpu.sync_copy(den_v, den_sh.at[sid])
        plsc.subcore_barrier()
        col = sid * rows_per_tile
        for sh, out in ((num_sh, num_out), (den_sh, den_out)):
            pltpu.sync_copy(sh.at[0, pl.ds(col, rows_per_tile)], acc_v)
            for t in range(1, NS):
                pltpu.sync_copy(sh.at[t, pl.ds(col, rows_per_tile)], tmp_v)

                def _acc(i, _):
                    acc_v[pl.ds(i * L, L)] = (
                        acc_v[pl.ds(i * L, L)] + tmp_v[pl.ds(i * L, L)])
                    return 0
                lax.fori_loop(0, rows_per_tile // L, _acc, 0)
            pltpu.sync_copy(acc_v, out.at[cid, pl.ds(col, rows_per_tile)])

    mesh = plsc.VectorSubcoreMesh(
        core_axis_name="c", subcore_axis_name="s", num_cores=NC, num_subcores=NS)
    return pl.kernel(
        body,
        out_type=[
            jax.ShapeDtypeStruct((NC, nseg), jnp.float32),
            jax.ShapeDtypeStruct((NC, nseg), jnp.float32),
        ],
        mesh=mesh,
        scratch_types=[
            pltpu.VMEM((C,), jnp.int32),
            pltpu.VMEM((C,), jnp.int32),
            pltpu.VMEM((nseg,), jnp.float32),
            pltpu.VMEM((nseg,), jnp.float32),
            pltpu.VMEM((nseg,), jnp.float32),
            pltpu.VMEM((nseg,), jnp.float32),
            pltpu.VMEM((L,), jnp.float32),
            pltpu.VMEM((rows_per_tile,), jnp.float32),
            pltpu.VMEM((rows_per_tile,), jnp.float32),
            pltpu.VMEM_SHARED((NS, nseg), jnp.float32),
            pltpu.VMEM_SHARED((NS, nseg), jnp.float32),
        ],
    )


def _tc_epilogue1(num, den, b1, wl2, wr2, nseg, h, block_rows):
    grid = nseg // block_rows

    def body(num_ref, den_ref, b1_ref, wl_ref, wr_ref, xl2_ref, xr2_ref):
        den_t = den_ref[0] + den_ref[1] + 1e-16
        hv = (num_ref[0] + num_ref[1]) / den_t[:, None] + b1_ref[...]
        xl2_ref[...] = jnp.sum(hv * wl_ref[...], axis=1)
        xr2_ref[...] = jnp.sum(hv * wr_ref[...], axis=1)

    return pl.pallas_call(
        body,
        grid=(grid,),
        in_specs=[
            pl.BlockSpec((NC, block_rows, h), lambda i: (0, i, 0)),
            pl.BlockSpec((NC, block_rows), lambda i: (0, i)),
            pl.BlockSpec((1, h), lambda i: (0, 0)),
            pl.BlockSpec((1, h), lambda i: (0, 0)),
            pl.BlockSpec((1, h), lambda i: (0, 0)),
        ],
        out_specs=[
            pl.BlockSpec((block_rows,), lambda i: (i,)),
            pl.BlockSpec((block_rows,), lambda i: (i,)),
        ],
        out_shape=[
            jax.ShapeDtypeStruct((nseg,), jnp.float32),
            jax.ShapeDtypeStruct((nseg,), jnp.float32),
        ],
    )(num, den, b1, wl2, wr2)


def _tc_epilogue2(num2, den2, b2, nseg):
    def body(n_ref, d_ref, b2_ref, o_ref):
        o_ref[...] = (n_ref[0] + n_ref[1]) / (d_ref[0] + d_ref[1] + 1e-16) \
            + b2_ref[0, 0]

    return pl.pallas_call(
        body,
        in_specs=[
            pl.BlockSpec(memory_space=pltpu.VMEM),
            pl.BlockSpec(memory_space=pltpu.VMEM),
            pl.BlockSpec(memory_space=pltpu.SMEM),
        ],
        out_specs=pl.BlockSpec(memory_space=pltpu.VMEM),
        out_shape=jax.ShapeDtypeStruct((nseg,), jnp.float32),
    )(num2, den2, b2)


@jax.jit
def kernel(x, edge_index, Wl1, Wr1, att1, b1, Wl2, Wr2, att2, b2):
    n, d = x.shape
    h = Wl1.shape[1]
    e = edge_index.shape[1]
    etot = e + n
    chunks = -(-etot // (NW * C))
    ep = chunks * NW * C
    nseg = -(-n // (NS * L)) * (NS * L)

    loops = jnp.arange(n, dtype=jnp.int32)
    pad = jnp.zeros((ep - etot,), jnp.int32)
    srcp = jnp.concatenate([edge_index[0].astype(jnp.int32), loops, pad])
    dstp = jnp.concatenate([edge_index[1].astype(jnp.int32), loops, pad])

    xl1, xr1 = _tc_transform(x, Wl1, Wr1, block_rows=2000)
    num1, den1 = _make_sc_layer1(nseg, chunks, etot, h)(
        xl1, xr1, srcp, dstp, att1)
    xl2, xr2 = _tc_epilogue1(
        num1, den1, b1.reshape(1, h),
        Wl2.reshape(1, h), Wr2.reshape(1, h), nseg, h, block_rows=2048)
    att2v = jnp.full((L,), att2[0], jnp.float32)
    num2, den2 = _make_sc_layer2(nseg, chunks, etot)(
        xl2, xr2, srcp, dstp, att2v)
    out = _tc_epilogue2(num2, den2, b2.reshape(1, 1), nseg)
    return out[:n, None]


# speedup vs baseline: 15.5828x; 15.5828x over previous
import jax
import jax.numpy as jnp
from jax import lax
from jax.experimental import pallas as pl
from jax.experimental.pallas import tpu as pltpu
from jax.experimental.pallas import tpu_sc as plsc

NC = 2
NS = 16
NW = NC * NS
L = 16
C = 64
DW = 16


def _tc_transform(x, Wl, Wr, block_rows):
    n, d = x.shape
    h = Wl.shape[1]
    grid = n // block_rows

    def body(x_ref, wl_ref, wr_ref, xl_ref, xr_ref):
        xv = x_ref[...]
        xl_ref[...] = jnp.dot(xv, wl_ref[...], preferred_element_type=jnp.float32)
        xr_ref[...] = jnp.dot(xv, wr_ref[...], preferred_element_type=jnp.float32)

    return pl.pallas_call(
        body,
        grid=(grid,),
        in_specs=[
            pl.BlockSpec((block_rows, d), lambda i: (i, 0)),
            pl.BlockSpec((d, h), lambda i: (0, 0)),
            pl.BlockSpec((d, h), lambda i: (0, 0)),
        ],
        out_specs=[
            pl.BlockSpec((block_rows, h), lambda i: (i, 0)),
            pl.BlockSpec((block_rows, h), lambda i: (i, 0)),
        ],
        out_shape=[
            jax.ShapeDtypeStruct((n, h), jnp.float32),
            jax.ShapeDtypeStruct((n, h), jnp.float32),
        ],
    )(x, Wl, Wr)


def _make_sc_layer1(nseg, chunks, etot, h):
    rows_per_tile = nseg // NS
    nvr = h // L

    def body(xl_hbm, xr_hbm, src_hbm, dst_hbm, att_hbm, rid_hbm,
             num_out, den_out,
             src_v, dst_v, xl_v, xr_v, den16_v, att_v,
             num_sh, den_sh, sem1, sem2):
        cid = lax.axis_index("c")
        sid = lax.axis_index("s")
        wid = cid * NS + sid
        zero16 = jnp.zeros((L,), jnp.float32)

        def _zrow(i, _):
            for r in range(nvr):
                xl_v[i, pl.ds(r * L, L)] = zero16
            den16_v[i, :] = zero16
            return 0
        lax.fori_loop(0, C, _zrow, 0)
        for t in range(-(-rows_per_tile // C)):
            base_r = sid * rows_per_tile + min(t * C, rows_per_tile - C)
            pltpu.sync_copy(rid_hbm.at[pl.ds(base_r, C)], src_v)
            pltpu.sync_copy(xl_v, num_sh.at[src_v])
            pltpu.sync_copy(den16_v, den_sh.at[src_v])

        pltpu.sync_copy(att_hbm, att_v)
        plsc.subcore_barrier()

        attv = [att_v[pl.ds(r * L, L)] for r in range(nvr)]

        def chunk_body(k, _):
            base = wid * (chunks * C) + k * C
            pltpu.sync_copy(src_hbm.at[pl.ds(base, C)], src_v)
            pltpu.sync_copy(dst_hbm.at[pl.ds(base, C)], dst_v)
            d1 = pltpu.async_copy(xl_hbm.at[src_v], xl_v, sem1)
            d2 = pltpu.async_copy(xr_hbm.at[dst_v], xr_v, sem2)
            d1.wait()
            d2.wait()

            def group_body(g, _):
                lane = lax.iota(jnp.int32, L)
                mask0 = lane == 0
                perms = [lane ^ s for s in (8, 4, 2, 1)]
                gb = base + g * L
                for j in range(L):
                    ej = g * L + j
                    a = [xl_v[ej, pl.ds(r * L, L)] for r in range(nvr)]
                    acc = zero16
                    for r in range(nvr):
                        v = a[r] + xr_v[ej, pl.ds(r * L, L)]
                        lr = jnp.maximum(v, 0.2 * v)
                        acc = acc + lr * attv[r]
                    for pm in perms:
                        acc = acc + acc[pm]
                    valid = jnp.where(gb + j < etot, 1.0, 0.0)
                    p = jnp.exp(acc) * valid
                    for r in range(nvr):
                        xl_v[ej, pl.ds(r * L, L)] = p * a[r]
                    den16_v[ej, :] = jnp.where(mask0, p, 0.0)
                return 0
            lax.fori_loop(0, C // L, group_body, 0)
            pltpu.sync_copy(xl_v, num_sh.at[dst_v], add=True)
            pltpu.sync_copy(den16_v, den_sh.at[dst_v], add=True)
            return 0
        lax.fori_loop(0, chunks, chunk_body, 0)

        plsc.subcore_barrier()
        for t in range(-(-rows_per_tile // C)):
            base_r = sid * rows_per_tile + min(t * C, rows_per_tile - C)
            pltpu.sync_copy(rid_hbm.at[pl.ds(base_r, C)], src_v)
            d1 = pltpu.async_copy(num_sh.at[src_v], xl_v, sem1)
            d2 = pltpu.async_copy(den_sh.at[src_v], den16_v, sem2)
            d1.wait()
            d2.wait()
            pltpu.sync_copy(xl_v, num_out.at[cid, pl.ds(base_r, C)])
            pltpu.sync_copy(den16_v, den_out.at[cid, pl.ds(base_r, C)])

    mesh = plsc.VectorSubcoreMesh(
        core_axis_name="c", subcore_axis_name="s", num_cores=NC, num_subcores=NS)
    return pl.kernel(
        body,
        out_type=[
            jax.ShapeDtypeStruct((NC, nseg, h), jnp.float32),
            jax.ShapeDtypeStruct((NC, nseg, DW), jnp.float32),
        ],
        mesh=mesh,
        scratch_types=[
            pltpu.VMEM((C,), jnp.int32),
            pltpu.VMEM((C,), jnp.int32),
            pltpu.VMEM((C, h), jnp.float32),
            pltpu.VMEM((C, h), jnp.float32),
            pltpu.VMEM((C, DW), jnp.float32),
            pltpu.VMEM((h,), jnp.float32),
            pltpu.VMEM_SHARED((nseg, h), jnp.float32),
            pltpu.VMEM_SHARED((nseg, DW), jnp.float32),
            pltpu.SemaphoreType.DMA,
            pltpu.SemaphoreType.DMA,
        ],
    )


def _make_sc_layer2(nseg, chunks, etot):
    rows_per_tile = nseg // NS

    def body(xl_hbm, xr_hbm, src_hbm, dst_hbm, att_hbm, rid_hbm,
             acc_out,
             src_v, dst_v, s_v, t_v, row_v, att_v,
             acc_sh, sem1, sem2):
        cid = lax.axis_index("c")
        sid = lax.axis_index("s")
        wid = cid * NS + sid
        zero16 = jnp.zeros((L,), jnp.float32)

        def _zrow(i, _):
            row_v[i, :] = zero16
            return 0
        lax.fori_loop(0, C, _zrow, 0)
        for t in range(-(-rows_per_tile // C)):
            base_r = sid * rows_per_tile + min(t * C, rows_per_tile - C)
            pltpu.sync_copy(rid_hbm.at[pl.ds(base_r, C)], src_v)
            pltpu.sync_copy(row_v, acc_sh.at[src_v])

        pltpu.sync_copy(att_hbm, att_v)
        plsc.subcore_barrier()

        def chunk_body(k, _):
            base = wid * (chunks * C) + k * C
            pltpu.sync_copy(src_hbm.at[pl.ds(base, C)], src_v)
            pltpu.sync_copy(dst_hbm.at[pl.ds(base, C)], dst_v)
            d1 = pltpu.async_copy(xl_hbm.at[src_v], s_v, sem1)
            d2 = pltpu.async_copy(xr_hbm.at[dst_v], t_v, sem2)
            d1.wait()
            d2.wait()
            att2 = att_v[...]

            def edge_body(j, _):
                lane = lax.iota(jnp.int32, L)
                mask0 = lane == 0
                mask1 = lane == 1
                s_row = s_v[j, :]
                t_row = t_v[j, :]
                v = s_row + t_row
                lr = jnp.maximum(v, 0.2 * v)
                p_row = jnp.exp(lr * att2)
                valid = jnp.where(base + j < etot, 1.0, 0.0)
                ps = jnp.full((L,), p_row[0], jnp.float32) * valid
                ss = jnp.full((L,), s_row[0], jnp.float32)
                w = jnp.where(mask0, ps * ss, jnp.where(mask1, ps, 0.0))
                row_v[j, :] = w
                return 0
            lax.fori_loop(0, C, edge_body, 0)
            pltpu.sync_copy(row_v, acc_sh.at[dst_v], add=True)
            return 0
        lax.fori_loop(0, chunks, chunk_body, 0)

        plsc.subcore_barrier()
        for t in range(-(-rows_per_tile // C)):
            base_r = sid * rows_per_tile + min(t * C, rows_per_tile - C)
            pltpu.sync_copy(rid_hbm.at[pl.ds(base_r, C)], src_v)
            pltpu.async_copy(acc_sh.at[src_v], row_v, sem1).wait()
            pltpu.sync_copy(row_v, acc_out.at[cid, pl.ds(base_r, C)])

    mesh = plsc.VectorSubcoreMesh(
        core_axis_name="c", subcore_axis_name="s", num_cores=NC, num_subcores=NS)
    return pl.kernel(
        body,
        out_type=jax.ShapeDtypeStruct((NC, nseg, DW), jnp.float32),
        mesh=mesh,
        compiler_params=pltpu.CompilerParams(use_tc_tiling_on_sc=False),
        scratch_types=[
            pltpu.VMEM((C,), jnp.int32),
            pltpu.VMEM((C,), jnp.int32),
            pltpu.VMEM((C, DW), jnp.float32),
            pltpu.VMEM((C, DW), jnp.float32),
            pltpu.VMEM((C, DW), jnp.float32),
            pltpu.VMEM((L,), jnp.float32),
            pltpu.VMEM_SHARED((nseg, DW), jnp.float32),
            pltpu.SemaphoreType.DMA,
            pltpu.SemaphoreType.DMA,
        ],
    )


def _tc_epilogue1(num, den, b1, wl2, wr2, nseg, h):
    def body(num_ref, den_ref, b1_ref, wl_ref, wr_ref, xl2_ref, xr2_ref):
        den_t = den_ref[0] + den_ref[1] + 1e-16
        hv = (num_ref[0] + num_ref[1]) / den_t[:, None] + b1_ref[...]
        xl2_ref[...] = jnp.sum(hv * wl_ref[...], axis=1)
        xr2_ref[...] = jnp.sum(hv * wr_ref[...], axis=1)

    return pl.pallas_call(
        body,
        in_specs=[
            pl.BlockSpec(memory_space=pltpu.VMEM),
            pl.BlockSpec(memory_space=pltpu.VMEM),
            pl.BlockSpec(memory_space=pltpu.VMEM),
            pl.BlockSpec(memory_space=pltpu.VMEM),
            pl.BlockSpec(memory_space=pltpu.VMEM),
        ],
        out_specs=[
            pl.BlockSpec(memory_space=pltpu.VMEM),
            pl.BlockSpec(memory_space=pltpu.VMEM),
        ],
        out_shape=[
            jax.ShapeDtypeStruct((nseg,), jnp.float32),
            jax.ShapeDtypeStruct((nseg,), jnp.float32),
        ],
    )(num, den, b1, wl2, wr2)


def _tc_epilogue2(num2, den2, b2, nseg):
    def body(n_ref, d_ref, b2_ref, o_ref):
        o_ref[...] = (n_ref[0] + n_ref[1]) / (d_ref[0] + d_ref[1] + 1e-16) \
            + b2_ref[0, 0]

    return pl.pallas_call(
        body,
        in_specs=[
            pl.BlockSpec(memory_space=pltpu.VMEM),
            pl.BlockSpec(memory_space=pltpu.VMEM),
            pl.BlockSpec(memory_space=pltpu.SMEM),
        ],
        out_specs=pl.BlockSpec(memory_space=pltpu.VMEM),
        out_shape=jax.ShapeDtypeStruct((nseg,), jnp.float32),
    )(num2, den2, b2)


@jax.jit
def kernel(x, edge_index, Wl1, Wr1, att1, b1, Wl2, Wr2, att2, b2):
    n, d = x.shape
    h = Wl1.shape[1]
    e = edge_index.shape[1]
    etot = e + n
    chunks = -(-etot // (NW * C))
    ep = chunks * NW * C
    nseg = -(-n // (NS * 8)) * (NS * 8)

    loops = jnp.arange(n, dtype=jnp.int32)
    pad = jnp.zeros((ep - etot,), jnp.int32)
    srcp = jnp.concatenate([edge_index[0].astype(jnp.int32), loops, pad])
    dstp = jnp.concatenate([edge_index[1].astype(jnp.int32), loops, pad])
    rowids = jnp.arange(nseg, dtype=jnp.int32)

    xl1, xr1 = _tc_transform(x, Wl1, Wr1, block_rows=2000)
    num1, den1w = _make_sc_layer1(nseg, chunks, etot, h)(
        xl1, xr1, srcp, dstp, att1, rowids)
    den1 = den1w[:, :, 0]
    xl2, xr2 = _tc_epilogue1(
        num1, den1, b1.reshape(1, h),
        Wl2.reshape(1, h), Wr2.reshape(1, h), nseg, h)
    att2v = jnp.full((L,), att2[0], jnp.float32)
    xl2w = jnp.pad(xl2[:, None], ((0, 0), (0, DW - 1)))
    xr2w = jnp.pad(xr2[:, None], ((0, 0), (0, DW - 1)))
    acc2 = _make_sc_layer2(nseg, chunks, etot)(xl2w, xr2w, srcp, dstp, att2v,
                                                rowids)
    out = _tc_epilogue2(acc2[:, :, 0], acc2[:, :, 1], b2.reshape(1, 1), nseg)
    return out[:n, None]


# speedup vs baseline: 18.1932x; 1.1675x over previous
import jax
import jax.numpy as jnp
from jax import lax
from jax.experimental import pallas as pl
from jax.experimental.pallas import tpu as pltpu
from jax.experimental.pallas import tpu_sc as plsc

NC = 2
NS = 16
NW = NC * NS
L = 16
C = 96
DW = 16


def _tc_transform(x, Wl, Wr, block_rows):
    n, d = x.shape
    h = Wl.shape[1]
    grid = n // block_rows

    def body(x_ref, wl_ref, wr_ref, xl_ref, xr_ref):
        xv = x_ref[...]
        xl_ref[...] = jnp.dot(xv, wl_ref[...], preferred_element_type=jnp.float32)
        xr_ref[...] = jnp.dot(xv, wr_ref[...], preferred_element_type=jnp.float32)

    return pl.pallas_call(
        body,
        grid=(grid,),
        in_specs=[
            pl.BlockSpec((block_rows, d), lambda i: (i, 0)),
            pl.BlockSpec((d, h), lambda i: (0, 0)),
            pl.BlockSpec((d, h), lambda i: (0, 0)),
        ],
        out_specs=[
            pl.BlockSpec((block_rows, h), lambda i: (i, 0)),
            pl.BlockSpec((block_rows, h), lambda i: (i, 0)),
        ],
        out_shape=[
            jax.ShapeDtypeStruct((n, h), jnp.float32),
            jax.ShapeDtypeStruct((n, h), jnp.float32),
        ],
    )(x, Wl, Wr)


def _make_sc_layer1(nseg, chunks, etot, h):
    rows_per_tile = nseg // NS
    nvr = h // L

    def body(xl_hbm, xr_hbm, src_hbm, dst_hbm, att_hbm, rid_hbm,
             num_out, den_out,
             src_v, dst_v, xl_v, xr_v, den16_v, att_v,
             num_sh, den_sh, sem1, sem2):
        cid = lax.axis_index("c")
        sid = lax.axis_index("s")
        wid = cid * NS + sid
        zero16 = jnp.zeros((L,), jnp.float32)

        def _zrow(i, _):
            for r in range(nvr):
                xl_v[i, pl.ds(r * L, L)] = zero16
            den16_v[i, :] = zero16
            return 0
        lax.fori_loop(0, C, _zrow, 0)
        for t in range(-(-rows_per_tile // C)):
            base_r = sid * rows_per_tile + min(t * C, rows_per_tile - C)
            pltpu.sync_copy(rid_hbm.at[pl.ds(base_r, C)], src_v)
            pltpu.sync_copy(xl_v, num_sh.at[src_v])
            pltpu.sync_copy(den16_v, den_sh.at[src_v])

        pltpu.sync_copy(att_hbm, att_v)
        plsc.subcore_barrier()

        attv = [att_v[pl.ds(r * L, L)] for r in range(nvr)]

        def chunk_body(k, _):
            base = wid * (chunks * C) + k * C
            pltpu.sync_copy(src_hbm.at[pl.ds(base, C)], src_v)
            pltpu.sync_copy(dst_hbm.at[pl.ds(base, C)], dst_v)
            d1 = pltpu.async_copy(xl_hbm.at[src_v], xl_v, sem1)
            d2 = pltpu.async_copy(xr_hbm.at[dst_v], xr_v, sem2)
            d1.wait()
            d2.wait()

            def group_body(g, _):
                lane = lax.iota(jnp.int32, L)
                mask0 = lane == 0
                perms = [lane ^ s for s in (8, 4, 2, 1)]
                gb = base + g * L
                for j in range(L):
                    ej = g * L + j
                    a = [xl_v[ej, pl.ds(r * L, L)] for r in range(nvr)]
                    acc = zero16
                    for r in range(nvr):
                        v = a[r] + xr_v[ej, pl.ds(r * L, L)]
                        lr = jnp.maximum(v, 0.2 * v)
                        acc = acc + lr * attv[r]
                    for pm in perms:
                        acc = acc + acc[pm]
                    valid = jnp.where(gb + j < etot, 1.0, 0.0)
                    p = jnp.exp(acc) * valid
                    for r in range(nvr):
                        xl_v[ej, pl.ds(r * L, L)] = p * a[r]
                    den16_v[ej, :] = jnp.where(mask0, p, 0.0)
                return 0
            lax.fori_loop(0, C // L, group_body, 0)
            pltpu.sync_copy(xl_v, num_sh.at[dst_v], add=True)
            pltpu.sync_copy(den16_v, den_sh.at[dst_v], add=True)
            return 0
        lax.fori_loop(0, chunks, chunk_body, 0)

        plsc.subcore_barrier()
        for t in range(-(-rows_per_tile // C)):
            base_r = sid * rows_per_tile + min(t * C, rows_per_tile - C)
            pltpu.sync_copy(rid_hbm.at[pl.ds(base_r, C)], src_v)
            d1 = pltpu.async_copy(num_sh.at[src_v], xl_v, sem1)
            d2 = pltpu.async_copy(den_sh.at[src_v], den16_v, sem2)
            d1.wait()
            d2.wait()
            pltpu.sync_copy(xl_v, num_out.at[cid, pl.ds(base_r, C)])
            pltpu.sync_copy(den16_v, den_out.at[cid, pl.ds(base_r, C)])

    mesh = plsc.VectorSubcoreMesh(
        core_axis_name="c", subcore_axis_name="s", num_cores=NC, num_subcores=NS)
    return pl.kernel(
        body,
        out_type=[
            jax.ShapeDtypeStruct((NC, nseg, h), jnp.float32),
            jax.ShapeDtypeStruct((NC, nseg, DW), jnp.float32),
        ],
        mesh=mesh,
        scratch_types=[
            pltpu.VMEM((C,), jnp.int32),
            pltpu.VMEM((C,), jnp.int32),
            pltpu.VMEM((C, h), jnp.float32),
            pltpu.VMEM((C, h), jnp.float32),
            pltpu.VMEM((C, DW), jnp.float32),
            pltpu.VMEM((h,), jnp.float32),
            pltpu.VMEM_SHARED((nseg, h), jnp.float32),
            pltpu.VMEM_SHARED((nseg, DW), jnp.float32),
            pltpu.SemaphoreType.DMA,
            pltpu.SemaphoreType.DMA,
        ],
    )


def _make_sc_layer2(nseg, chunks, etot):
    rows_per_tile = nseg // NS

    def body(xl_hbm, xr_hbm, src_hbm, dst_hbm, att_hbm, rid_hbm,
             acc_out,
             src_v, dst_v, s_v, t_v, row_v, att_v,
             acc_sh, sem1, sem2):
        cid = lax.axis_index("c")
        sid = lax.axis_index("s")
        wid = cid * NS + sid
        zero16 = jnp.zeros((L,), jnp.float32)

        def _zrow(i, _):
            row_v[i, :] = zero16
            return 0
        lax.fori_loop(0, C, _zrow, 0)
        for t in range(-(-rows_per_tile // C)):
            base_r = sid * rows_per_tile + min(t * C, rows_per_tile - C)
            pltpu.sync_copy(rid_hbm.at[pl.ds(base_r, C)], src_v)
            pltpu.sync_copy(row_v, acc_sh.at[src_v])

        pltpu.sync_copy(att_hbm, att_v)
        plsc.subcore_barrier()

        def chunk_body(k, _):
            base = wid * (chunks * C) + k * C
            pltpu.sync_copy(src_hbm.at[pl.ds(base, C)], src_v)
            pltpu.sync_copy(dst_hbm.at[pl.ds(base, C)], dst_v)
            d1 = pltpu.async_copy(xl_hbm.at[src_v], s_v, sem1)
            d2 = pltpu.async_copy(xr_hbm.at[dst_v], t_v, sem2)
            d1.wait()
            d2.wait()
            att2 = att_v[...]

            def edge_body(j, _):
                lane = lax.iota(jnp.int32, L)
                mask0 = lane == 0
                mask1 = lane == 1
                s_row = s_v[j, :]
                t_row = t_v[j, :]
                v = s_row + t_row
                lr = jnp.maximum(v, 0.2 * v)
                p_row = jnp.exp(lr * att2)
                valid = jnp.where(base + j < etot, 1.0, 0.0)
                ps = jnp.full((L,), p_row[0], jnp.float32) * valid
                ss = jnp.full((L,), s_row[0], jnp.float32)
                w = jnp.where(mask0, ps * ss, jnp.where(mask1, ps, 0.0))
                row_v[j, :] = w
                return 0
            lax.fori_loop(0, C, edge_body, 0)
            pltpu.sync_copy(row_v, acc_sh.at[dst_v], add=True)
            return 0
        lax.fori_loop(0, chunks, chunk_body, 0)

        plsc.subcore_barrier()
        for t in range(-(-rows_per_tile // C)):
            base_r = sid * rows_per_tile + min(t * C, rows_per_tile - C)
            pltpu.sync_copy(rid_hbm.at[pl.ds(base_r, C)], src_v)
            pltpu.async_copy(acc_sh.at[src_v], row_v, sem1).wait()
            pltpu.sync_copy(row_v, acc_out.at[cid, pl.ds(base_r, C)])

    mesh = plsc.VectorSubcoreMesh(
        core_axis_name="c", subcore_axis_name="s", num_cores=NC, num_subcores=NS)
    return pl.kernel(
        body,
        out_type=jax.ShapeDtypeStruct((NC, nseg, DW), jnp.float32),
        mesh=mesh,
        compiler_params=pltpu.CompilerParams(use_tc_tiling_on_sc=False),
        scratch_types=[
            pltpu.VMEM((C,), jnp.int32),
            pltpu.VMEM((C,), jnp.int32),
            pltpu.VMEM((C, DW), jnp.float32),
            pltpu.VMEM((C, DW), jnp.float32),
            pltpu.VMEM((C, DW), jnp.float32),
            pltpu.VMEM((L,), jnp.float32),
            pltpu.VMEM_SHARED((nseg, DW), jnp.float32),
            pltpu.SemaphoreType.DMA,
            pltpu.SemaphoreType.DMA,
        ],
    )


def _tc_epilogue1(num, den, b1, wl2, wr2, nseg, h):
    def body(num_ref, den_ref, b1_ref, wl_ref, wr_ref, xl2_ref, xr2_ref):
        den_t = den_ref[0] + den_ref[1] + 1e-16
        hv = (num_ref[0] + num_ref[1]) / den_t[:, None] + b1_ref[...]
        xl2_ref[...] = jnp.sum(hv * wl_ref[...], axis=1)
        xr2_ref[...] = jnp.sum(hv * wr_ref[...], axis=1)

    return pl.pallas_call(
        body,
        in_specs=[
            pl.BlockSpec(memory_space=pltpu.VMEM),
            pl.BlockSpec(memory_space=pltpu.VMEM),
            pl.BlockSpec(memory_space=pltpu.VMEM),
            pl.BlockSpec(memory_space=pltpu.VMEM),
            pl.BlockSpec(memory_space=pltpu.VMEM),
        ],
        out_specs=[
            pl.BlockSpec(memory_space=pltpu.VMEM),
            pl.BlockSpec(memory_space=pltpu.VMEM),
        ],
        out_shape=[
            jax.ShapeDtypeStruct((nseg,), jnp.float32),
            jax.ShapeDtypeStruct((nseg,), jnp.float32),
        ],
    )(num, den, b1, wl2, wr2)


def _tc_epilogue2(num2, den2, b2, nseg):
    def body(n_ref, d_ref, b2_ref, o_ref):
        o_ref[...] = (n_ref[0] + n_ref[1]) / (d_ref[0] + d_ref[1] + 1e-16) \
            + b2_ref[0, 0]

    return pl.pallas_call(
        body,
        in_specs=[
            pl.BlockSpec(memory_space=pltpu.VMEM),
            pl.BlockSpec(memory_space=pltpu.VMEM),
            pl.BlockSpec(memory_space=pltpu.SMEM),
        ],
        out_specs=pl.BlockSpec(memory_space=pltpu.VMEM),
        out_shape=jax.ShapeDtypeStruct((nseg,), jnp.float32),
    )(num2, den2, b2)


@jax.jit
def kernel(x, edge_index, Wl1, Wr1, att1, b1, Wl2, Wr2, att2, b2):
    n, d = x.shape
    h = Wl1.shape[1]
    e = edge_index.shape[1]
    etot = e + n
    chunks = -(-etot // (NW * C))
    ep = chunks * NW * C
    nseg = -(-n // (NS * 8)) * (NS * 8)

    loops = jnp.arange(n, dtype=jnp.int32)
    pad = jnp.zeros((ep - etot,), jnp.int32)
    srcp = jnp.concatenate([edge_index[0].astype(jnp.int32), loops, pad])
    dstp = jnp.concatenate([edge_index[1].astype(jnp.int32), loops, pad])
    rowids = jnp.arange(nseg, dtype=jnp.int32)

    xl1, xr1 = _tc_transform(x, Wl1, Wr1, block_rows=2000)
    num1, den1w = _make_sc_layer1(nseg, chunks, etot, h)(
        xl1, xr1, srcp, dstp, att1, rowids)
    den1 = den1w[:, :, 0]
    xl2, xr2 = _tc_epilogue1(
        num1, den1, b1.reshape(1, h),
        Wl2.reshape(1, h), Wr2.reshape(1, h), nseg, h)
    att2v = jnp.full((L,), att2[0], jnp.float32)
    xl2w = jnp.pad(xl2[:, None], ((0, 0), (0, DW - 1)))
    xr2w = jnp.pad(xr2[:, None], ((0, 0), (0, DW - 1)))
    acc2 = _make_sc_layer2(nseg, chunks, etot)(xl2w, xr2w, srcp, dstp, att2v,
                                                rowids)
    out = _tc_epilogue2(acc2[:, :, 0], acc2[:, :, 1], b2.reshape(1, 1), nseg)
    return out[:n, None]
